# Initial kernel scaffold; baseline (speedup 1.0000x reference)
import jax
import jax.numpy as jnp
from jax.experimental import pallas as pl


def kernel(indices, tables, bias):
    raise NotImplementedError("write your pallas kernel here")


# speedup vs baseline: 1.0169x; 1.0169x over previous
import functools

import jax
import jax.numpy as jnp
from jax import lax
from jax.experimental import pallas as pl
from jax.experimental.pallas import tpu as pltpu
from jax.experimental.pallas import tpu_sc as plsc

_L = 16
_NC = 2
_NS = 16


@functools.cache
def _make_sc_kernel(F, V, B):
    NW = _NC * _NS
    bpw = B // NW
    nsl = bpw // _L
    n = F * bpw

    mesh = plsc.VectorSubcoreMesh(core_axis_name="c", subcore_axis_name="s")

    @functools.partial(
        pl.kernel,
        mesh=mesh,
        out_type=jax.ShapeDtypeStruct((B,), jnp.float32),
        scratch_types=[
            pltpu.VMEM((n,), jnp.int32),
            pltpu.VMEM((n,), jnp.float32),
            pltpu.VMEM((bpw,), jnp.float32),
            pltpu.VMEM((_L,), jnp.float32),
            pltpu.SemaphoreType.DMA,
        ],
    )
    def k(idx_hbm, table_hbm, bias_hbm, out_hbm, idx_v, gat_v, out_v, bias_v,
          sem):
        wid = lax.axis_index("s") * _NC + lax.axis_index("c")
        base = wid * bpw
        pltpu.sync_copy(bias_hbm, bias_v)
        pltpu.sync_copy(idx_hbm.at[wid], idx_v)

        def off_body(i, _):
            off = (i // (bpw // _L)) * V
            sl = pl.ds(i * _L, _L)
            idx_v[sl] = idx_v[sl] + off
            return 0

        lax.fori_loop(0, F * nsl, off_body, 0)

        pltpu.async_copy(table_hbm.at[idx_v], gat_v, sem).wait()

        bias_vec = bias_v[...]

        def sum_body(s, _):
            acc = bias_vec
            for f in range(F):
                acc = acc + gat_v[pl.ds(f * bpw + s * _L, _L)]
            out_v[pl.ds(s * _L, _L)] = 1.0 / (1.0 + jnp.exp(-acc))
            return 0

        lax.fori_loop(0, nsl, sum_body, 0)

        pltpu.sync_copy(out_v, out_hbm.at[pl.ds(base, bpw)])

    return k


@jax.jit
def kernel(indices, tables, bias):
    F, V = tables.shape
    B = indices.shape[0]
    NW = _NC * _NS
    bpw = B // NW
    idx_w = (
        indices.astype(jnp.int32).T.reshape(F, NW, bpw)
        .transpose(1, 0, 2).reshape(NW, F * bpw)
    )
    flat_table = tables.reshape(-1)
    bias16 = jnp.broadcast_to(bias.astype(jnp.float32), (_L,))
    return _make_sc_kernel(F, V, B)(idx_w, flat_table, bias16)


# speedup vs baseline: 19.5090x; 19.1844x over previous
import functools

import jax
import jax.numpy as jnp
from jax import lax
from jax.experimental import pallas as pl
from jax.experimental.pallas import tpu as pltpu
from jax.experimental.pallas import tpu_sc as plsc

_L = 16
_NC = 2
_NS = 16


@functools.cache
def _make_detile(r0, r1, V, VP):
    NR = r1 - r0
    body_len = (V // 128) * 128
    tail = V - body_len
    C = body_len // 2
    nfull = NR // 8
    nlast = NR - 8 * nfull
    units = [(g, k) for g in range(nfull + (1 if nlast else 0))
             for k in range(2)]

    def body(in_ref, tails_ref, out_ref, buf, rsem, wsem, tsem):
        tail_cps = []
        if tail:
            for f in range(NR):
                tail_cps.append(pltpu.make_async_copy(
                    tails_ref.at[pl.ds(f * 128, 128)],
                    out_ref.at[pl.ds(f * VP + body_len, 128)], tsem))
        for c in tail_cps:
            c.start()

        def unit_reads(u, g, k):
            if g < nfull:
                return [pltpu.make_async_copy(
                    in_ref.at[pl.ds(r0 + 8 * g, 8), pl.ds(k * C, C)],
                    buf.at[u % 2], rsem[u % 2])]
            return [pltpu.make_async_copy(
                in_ref.at[pl.ds(r0 + 8 * nfull + j, 1), pl.ds(k * C, C)],
                buf.at[u % 2, pl.ds(j, 1), :], rsem[u % 2])
                for j in range(nlast)]

        reads = [unit_reads(u, g, k) for u, (g, k) in enumerate(units)]
        for r in reads[0]:
            r.start()
        pending = {0: [], 1: []}
        for u, (g, k) in enumerate(units):
            if u + 1 < len(units):
                for w in pending[(u + 1) % 2]:
                    w.wait()
                pending[(u + 1) % 2] = []
                for r in reads[u + 1]:
                    r.start()
            for r in reads[u]:
                r.wait()
            nrows = 8 if g < nfull else nlast
            for r in range(nrows):
                w = pltpu.make_async_copy(
                    buf.at[u % 2, r],
                    out_ref.at[pl.ds((8 * g + r) * VP + k * C, C)],
                    wsem[u % 2])
                w.start()
                pending[u % 2].append(w)
        for b in (0, 1):
            for w in pending[b]:
                w.wait()
        for c in tail_cps:
            c.wait()

    return pl.pallas_call(
        body,
        in_specs=[pl.BlockSpec(memory_space=pl.ANY),
                  pl.BlockSpec(memory_space=pl.ANY)],
        out_specs=pl.BlockSpec(memory_space=pl.ANY),
        out_shape=jax.ShapeDtypeStruct((NR * VP,), jnp.float32),
        scratch_shapes=[
            pltpu.VMEM((2, 8, C), jnp.float32),
            [pltpu.SemaphoreType.DMA for _ in range(2)],
            [pltpu.SemaphoreType.DMA for _ in range(2)],
            pltpu.SemaphoreType.DMA,
        ],
    )


@functools.cache
def _make_sc_gather(NF, off_fields, FT, VP, B, final):
    NW = _NC * _NS
    bpw = B // NW
    nsl = bpw // _L
    n = NF * bpw

    mesh = plsc.VectorSubcoreMesh(core_axis_name="c", subcore_axis_name="s")

    scratch = [
        pltpu.VMEM((n,), jnp.int32),
        pltpu.VMEM((n,), jnp.float32),
        pltpu.VMEM((bpw,), jnp.float32),
        pltpu.VMEM((_L,), jnp.float32),
        pltpu.SemaphoreType.DMA,
    ]
    if final:
        scratch.insert(3, pltpu.VMEM((bpw,), jnp.float32))

    @functools.partial(
        pl.kernel,
        mesh=mesh,
        out_type=jax.ShapeDtypeStruct((B,), jnp.float32),
        scratch_types=scratch,
    )
    def k(idx_hbm, table_hbm, bias_hbm, *rest):
        if final:
            (part_hbm, out_hbm, idx_v, gat_v, out_v, part_v, bias_v,
             sem) = rest
        else:
            (out_hbm, idx_v, gat_v, out_v, bias_v, sem) = rest
        wid = lax.axis_index("s") * _NC + lax.axis_index("c")
        base = wid * bpw
        pltpu.sync_copy(bias_hbm, bias_v)
        pltpu.sync_copy(idx_hbm.at[wid, pl.ds(off_fields * bpw, n)], idx_v)
        if final:
            pltpu.sync_copy(part_hbm.at[pl.ds(base, bpw)], part_v)

        def off_body(i, _):
            off = (i // nsl) * VP
            sl = pl.ds(i * _L, _L)
            idx_v[sl] = idx_v[sl] + off
            return 0

        lax.fori_loop(0, NF * nsl, off_body, 0)

        pltpu.async_copy(table_hbm.at[idx_v], gat_v, sem).wait()

        bias_vec = bias_v[...]

        def sum_body(s, _):
            sl = pl.ds(s * _L, _L)
            if final:
                acc = bias_vec + part_v[sl]
                f0 = 0
            else:
                acc = gat_v[sl]
                f0 = 1
            for f in range(f0, NF):
                acc = acc + gat_v[pl.ds(f * bpw + s * _L, _L)]
            if final:
                out_v[sl] = 1.0 / (1.0 + jnp.exp(-acc))
            else:
                out_v[sl] = acc
            return 0

        lax.fori_loop(0, nsl, sum_body, 0)

        pltpu.sync_copy(out_v, out_hbm.at[pl.ds(base, bpw)])

    return k


_F_SPLIT = 16


@jax.jit
def kernel(indices, tables, bias):
    F, V = tables.shape
    B = indices.shape[0]
    NW = _NC * _NS
    bpw = B // NW
    VP = ((V + 127) // 128) * 128
    body_len = (V // 128) * 128
    FA = min(_F_SPLIT, F)
    FB = F - FA

    idx_w = (
        indices.astype(jnp.int32).T.reshape(F, NW, bpw)
        .transpose(1, 0, 2).reshape(NW, F * bpw)
    )
    tails = jnp.pad(tables[:, body_len:], ((0, 0), (0, 128 - (V - body_len)))
                    ).reshape(-1)
    bias16 = jnp.broadcast_to(bias.astype(jnp.float32), (_L,))

    flat_a = _make_detile(0, FA, V, VP)(tables, tails[:FA * 128])
    part = _make_sc_gather(FA, 0, F, VP, B, False)(idx_w, flat_a, bias16)
    if FB:
        flat_b = _make_detile(FA, F, V, VP)(tables, tails[FA * 128:])
        return _make_sc_gather(FB, FA, F, VP, B, True)(
            idx_w, flat_b, bias16, part)
    return jax.nn.sigmoid(part + bias[0])
